# baseline (speedup 1.0000x reference)
import functools

import jax
import jax.numpy as jnp
from jax import lax
from jax.experimental import pallas as pl
from jax.experimental.pallas import tpu as pltpu
from jax.experimental.pallas import tpu_sc as plsc

VOCAB = 100000
EMB = 64
NL = 9

NC = 2
NS = 16
NW = NC * NS

CHUNK = 1600


def _fold_body(emb_ref, w_ref, b_ref, out_ref):
    out_ref[...] = (
        jnp.dot(emb_ref[...], w_ref[...], preferred_element_type=jnp.float32)
        + b_ref[...]
    )


def _fold_table(embedding, fc_w, fc_b):
    w = jnp.zeros((EMB, 16), jnp.float32).at[:, :NL].set(fc_w)
    b = jnp.zeros((1, 16), jnp.float32).at[0, :NL].set(fc_b)
    blk = 2000
    return pl.pallas_call(
        _fold_body,
        grid=(VOCAB // blk,),
        in_specs=[
            pl.BlockSpec((blk, EMB), lambda i: (i, 0)),
            pl.BlockSpec((EMB, 16), lambda i: (0, 0)),
            pl.BlockSpec((1, 16), lambda i: (0, 0)),
        ],
        out_specs=pl.BlockSpec((blk, 16), lambda i: (i, 0)),
        out_shape=jax.ShapeDtypeStruct((VOCAB, 16), jnp.float32),
    )(embedding, w, b)


DPAD = 16


def _make_gather(ntok):
    per_w = ntok // NW
    steps = per_w // CHUNK
    assert per_w % CHUNK == 0 and steps % 2 == 0
    groups = CHUNK // 16
    pair_vals = 2 * CHUNK * NL
    assert pair_vals % 128 == 0
    pair_rows = pair_vals // 128
    out_rows = ntok * NL // 128

    @functools.partial(
        pl.kernel,
        mesh=plsc.VectorSubcoreMesh(core_axis_name="c", subcore_axis_name="s"),
        out_type=jax.ShapeDtypeStruct((out_rows, 128), jnp.float32),
        scratch_types=[
            pltpu.VMEM((CHUNK,), jnp.int32),
            pltpu.VMEM((CHUNK,), jnp.int32),
            pltpu.VMEM((CHUNK, DPAD), jnp.float32),
            pltpu.VMEM((CHUNK, DPAD), jnp.float32),
            pltpu.VMEM((pair_rows, 128), jnp.float32),
            pltpu.VMEM((pair_rows, 128), jnp.float32),
            pltpu.SemaphoreType.DMA,
            pltpu.SemaphoreType.DMA,
            pltpu.SemaphoreType.DMA,
            pltpu.SemaphoreType.DMA,
        ],
        compiler_params=pltpu.CompilerParams(
            use_tc_tiling_on_sc=False,
            needs_layout_passes=False,
            disable_bounds_checks=True,
        ),
    )
    def gather(tbl_hbm, idx_hbm, out_hbm, i0, i1, r0, r1, c0, c1,
               sg0, sg1, ss0, ss1):
        wid = lax.axis_index("s") * NC + lax.axis_index("c")
        base = wid * per_w
        idx = (i0, i1)
        rows = (r0, r1)
        comp = (c0, c1)
        gsem = (sg0, sg1)
        ssem = (ss0, ss1)

        lane = lax.iota(jnp.int32, 16)
        rq = [(k * 16 + lane) // NL for k in range(NL)]
        cq = [(k * 16 + lane) % NL for k in range(NL)]

        def compact(r16, r9, half):
            def body(g, carry):
                g16 = g * 16
                s0 = half * (CHUNK * NL) + g * (16 * NL)
                for k in range(NL):
                    v = plsc.load_gather(r16, [g16 + rq[k], cq[k]])
                    s = s0 + k * 16
                    r9[s // 128, pl.ds(s % 128, 16)] = v
                return carry

            lax.fori_loop(0, groups, body, 0)

        pltpu.sync_copy(idx_hbm.at[pl.ds(base, CHUNK)], idx[0])
        gathers = [pltpu.async_copy(tbl_hbm.at[idx[0]], rows[0], gsem[0])]
        stores = [None, None]
        for i in range(steps):
            b = i % 2
            nb = (i + 1) % 2
            if i + 1 < steps:
                pltpu.sync_copy(
                    idx_hbm.at[pl.ds(base + (i + 1) * CHUNK, CHUNK)], idx[nb]
                )
                gathers.append(
                    pltpu.async_copy(tbl_hbm.at[idx[nb]], rows[nb], gsem[nb])
                )
            gathers[i].wait()
            pb = (i // 2) % 2
            if i % 2 == 0 and stores[pb] is not None:
                stores[pb].wait()
            compact(rows[b], comp[pb], i % 2)
            if i % 2 == 1:
                row0 = (base * NL + (i - 1) * CHUNK * NL) // 128
                stores[pb] = pltpu.async_copy(
                    comp[pb],
                    out_hbm.at[pl.ds(row0, pair_rows), :],
                    ssem[pb],
                )
        stores[0].wait()
        stores[1].wait()

    return gather


def kernel(inputs_ids, input_lens, embedding, fc_w, fc_b):
    del input_lens
    b, l = inputs_ids.shape
    tbl = _fold_table(embedding, fc_w, fc_b)
    idx = inputs_ids.reshape(-1).astype(jnp.int32)
    out = _make_gather(b * l)(tbl, idx)
    return out.reshape(b, l, NL)


# speedup vs baseline: 1.3541x; 1.3541x over previous
import functools

import jax
import jax.numpy as jnp
from jax import lax
from jax.experimental import pallas as pl
from jax.experimental.pallas import tpu as pltpu
from jax.experimental.pallas import tpu_sc as plsc

VOCAB = 100000
EMB = 64
NL = 9

NC = 2
NS = 16
NW = NC * NS

CHUNK = 3200


def _fold_body(emb_ref, w_ref, b_ref, out_ref):
    out_ref[...] = (
        jnp.dot(emb_ref[...], w_ref[...], preferred_element_type=jnp.float32)
        + b_ref[...]
    )


def _fold_table(embedding, fc_w, fc_b):
    w = jnp.zeros((EMB, 16), jnp.float32).at[:, :NL].set(fc_w)
    b = jnp.zeros((1, 16), jnp.float32).at[0, :NL].set(fc_b)
    blk = 2000
    return pl.pallas_call(
        _fold_body,
        grid=(VOCAB // blk,),
        in_specs=[
            pl.BlockSpec((blk, EMB), lambda i: (i, 0)),
            pl.BlockSpec((EMB, 16), lambda i: (0, 0)),
            pl.BlockSpec((1, 16), lambda i: (0, 0)),
        ],
        out_specs=pl.BlockSpec((blk, 16), lambda i: (i, 0)),
        out_shape=jax.ShapeDtypeStruct((VOCAB, 16), jnp.float32),
    )(embedding, w, b)


DPAD = 16


def _make_gather(ntok):
    per_w = ntok // NW
    steps = per_w // CHUNK
    assert per_w % CHUNK == 0

    @functools.partial(
        pl.kernel,
        mesh=plsc.VectorSubcoreMesh(core_axis_name="c", subcore_axis_name="s"),
        out_type=jax.ShapeDtypeStruct((ntok, DPAD), jnp.float32),
        scratch_types=[
            pltpu.VMEM((CHUNK,), jnp.int32),
            pltpu.VMEM((CHUNK,), jnp.int32),
            pltpu.VMEM((CHUNK, DPAD), jnp.float32),
            pltpu.VMEM((CHUNK, DPAD), jnp.float32),
            pltpu.SemaphoreType.DMA,
            pltpu.SemaphoreType.DMA,
            pltpu.SemaphoreType.DMA,
            pltpu.SemaphoreType.DMA,
        ],
        compiler_params=pltpu.CompilerParams(
            use_tc_tiling_on_sc=False,
            needs_layout_passes=False,
            disable_bounds_checks=True,
        ),
    )
    def gather(tbl_hbm, idx_hbm, out_hbm, i0, i1, r0, r1, sg0, sg1, ss0, ss1):
        wid = lax.axis_index("s") * NC + lax.axis_index("c")
        base = wid * per_w
        idx = (i0, i1)
        rows = (r0, r1)
        gsem = (sg0, sg1)
        ssem = (ss0, ss1)

        pltpu.sync_copy(idx_hbm.at[pl.ds(base, CHUNK)], idx[0])
        gathers = [pltpu.async_copy(tbl_hbm.at[idx[0]], rows[0], gsem[0])]
        stores = [None, None]
        for i in range(steps):
            b = i % 2
            nb = (i + 1) % 2
            if i + 1 < steps:
                pltpu.sync_copy(
                    idx_hbm.at[pl.ds(base + (i + 1) * CHUNK, CHUNK)], idx[nb]
                )
                if stores[nb] is not None:
                    stores[nb].wait()
                gathers.append(
                    pltpu.async_copy(tbl_hbm.at[idx[nb]], rows[nb], gsem[nb])
                )
            gathers[i].wait()
            stores[b] = pltpu.async_copy(
                rows[b],
                out_hbm.at[pl.ds(base + i * CHUNK, CHUNK), :],
                ssem[b],
            )
        stores[(steps - 1) % 2].wait()
        stores[steps % 2].wait()

    return gather


def kernel(inputs_ids, input_lens, embedding, fc_w, fc_b):
    del input_lens
    b, l = inputs_ids.shape
    tbl = _fold_table(embedding, fc_w, fc_b)
    idx = inputs_ids.reshape(-1).astype(jnp.int32)
    out = _make_gather(b * l)(tbl, idx)
    return out[:, :NL].reshape(b, l, NL)


# speedup vs baseline: 1.3549x; 1.0006x over previous
import functools

import jax
import jax.numpy as jnp
from jax import lax
from jax.experimental import pallas as pl
from jax.experimental.pallas import tpu as pltpu
from jax.experimental.pallas import tpu_sc as plsc

VOCAB = 100000
EMB = 64
NL = 9

NC = 2
NS = 16
NW = NC * NS

CHUNK = 1600


def _fold_body(emb_ref, w_ref, b_ref, out_ref):
    out_ref[...] = (
        jnp.dot(emb_ref[...], w_ref[...], preferred_element_type=jnp.float32)
        + b_ref[...]
    )


def _fold_table(embedding, fc_w, fc_b):
    w = jnp.zeros((EMB, 16), jnp.float32).at[:, :NL].set(fc_w)
    b = jnp.zeros((1, 16), jnp.float32).at[0, :NL].set(fc_b)
    blk = 2000
    return pl.pallas_call(
        _fold_body,
        grid=(VOCAB // blk,),
        in_specs=[
            pl.BlockSpec((blk, EMB), lambda i: (i, 0)),
            pl.BlockSpec((EMB, 16), lambda i: (0, 0)),
            pl.BlockSpec((1, 16), lambda i: (0, 0)),
        ],
        out_specs=pl.BlockSpec((blk, 16), lambda i: (i, 0)),
        out_shape=jax.ShapeDtypeStruct((VOCAB, 16), jnp.float32),
    )(embedding, w, b)


DPAD = 16


def _make_gather(ntok):
    per_w = ntok // NW
    steps = per_w // CHUNK
    assert per_w % CHUNK == 0

    @functools.partial(
        pl.kernel,
        mesh=plsc.VectorSubcoreMesh(core_axis_name="c", subcore_axis_name="s"),
        out_type=jax.ShapeDtypeStruct((ntok, DPAD), jnp.float32),
        scratch_types=[
            pltpu.VMEM((CHUNK,), jnp.int32),
            pltpu.VMEM((CHUNK,), jnp.int32),
            pltpu.VMEM((CHUNK, DPAD), jnp.float32),
            pltpu.VMEM((CHUNK, DPAD), jnp.float32),
            pltpu.SemaphoreType.DMA,
            pltpu.SemaphoreType.DMA,
            pltpu.SemaphoreType.DMA,
            pltpu.SemaphoreType.DMA,
        ],
        compiler_params=pltpu.CompilerParams(
            use_tc_tiling_on_sc=False,
            needs_layout_passes=False,
            disable_bounds_checks=True,
        ),
    )
    def gather(tbl_hbm, idx_hbm, out_hbm, i0, i1, r0, r1, sg0, sg1, ss0, ss1):
        wid = lax.axis_index("s") * NC + lax.axis_index("c")
        base = wid * per_w
        idx = (i0, i1)
        rows = (r0, r1)
        gsem = (sg0, sg1)
        ssem = (ss0, ss1)

        pltpu.sync_copy(idx_hbm.at[pl.ds(base, CHUNK)], idx[0])
        gathers = [pltpu.async_copy(tbl_hbm.at[idx[0]], rows[0], gsem[0])]
        stores = [None, None]
        for i in range(steps):
            b = i % 2
            nb = (i + 1) % 2
            if i + 1 < steps:
                pltpu.sync_copy(
                    idx_hbm.at[pl.ds(base + (i + 1) * CHUNK, CHUNK)], idx[nb]
                )
                if stores[nb] is not None:
                    stores[nb].wait()
                gathers.append(
                    pltpu.async_copy(tbl_hbm.at[idx[nb]], rows[nb], gsem[nb])
                )
            gathers[i].wait()
            stores[b] = pltpu.async_copy(
                rows[b],
                out_hbm.at[pl.ds(base + i * CHUNK, CHUNK), :],
                ssem[b],
            )
        stores[(steps - 1) % 2].wait()
        stores[steps % 2].wait()

    return gather


def kernel(inputs_ids, input_lens, embedding, fc_w, fc_b):
    del input_lens
    b, l = inputs_ids.shape
    tbl = _fold_table(embedding, fc_w, fc_b)
    idx = inputs_ids.reshape(-1).astype(jnp.int32)
    out = _make_gather(b * l)(tbl, idx)
    return out[:, :NL].reshape(b, l, NL)
